# baseline (speedup 1.0000x reference)
import jax
import jax.numpy as jnp
from jax import lax
from jax.experimental import pallas as pl
from jax.experimental.pallas import tpu as pltpu
from jax.experimental.pallas import tpu_sc as plsc

B = 16384
D = 32
DP = 128
L_CTX = 50
L_ITM = 20
L_TOT = L_CTX + L_ITM
NC = 2
NS = 16
NW = NC * NS
BW = B // NW
CB = 4
NCHUNK = BW // CB
NQ = NCHUNK // 4
LANES = 16


def _sc_body(ctx_hbm, itm_hbm, w_hbm, table_hbm, out_hbm,
             cidx0, iidx0, w0, rows0, cidx1, iidx1, w1, rows1,
             out_v, sem0, sem1):
    wid = lax.axis_index("s") * NC + lax.axis_index("c")
    base_b = wid * BW
    bufs = ((cidx0, iidx0, w0, rows0, sem0),
            (cidx1, iidx1, w1, rows1, sem1))

    def fire(c, buf):
        cidx_v, iidx_v, w_v, rows_v, sem = buf
        b0 = base_b + c * CB
        pltpu.sync_copy(ctx_hbm.at[pl.ds(b0, CB)], cidx_v)
        pltpu.sync_copy(itm_hbm.at[pl.ds(b0, CB)], iidx_v)
        pltpu.sync_copy(w_hbm.at[pl.ds(b0, CB)], w_v)
        for i in range(CB):
            pltpu.async_copy(table_hbm.at[cidx_v.at[i]],
                             rows_v.at[pl.ds(i * L_TOT, L_CTX)], sem)
            pltpu.async_copy(table_hbm.at[iidx_v.at[i]],
                             rows_v.at[pl.ds(i * L_TOT + L_CTX, L_ITM)], sem)

    def drain(buf):
        cidx_v, iidx_v, w_v, rows_v, sem = buf
        for i in range(CB):
            pltpu.make_async_copy(table_hbm.at[cidx_v.at[i]],
                                  rows_v.at[pl.ds(i * L_TOT, L_CTX)],
                                  sem).wait()
            pltpu.make_async_copy(table_hbm.at[iidx_v.at[i]],
                                  rows_v.at[pl.ds(i * L_TOT + L_CTX, L_ITM)],
                                  sem).wait()

    lane_iota = lax.iota(jnp.int32, LANES)

    def compute(c, k, buf, dots):
        cidx_v, iidx_v, w_v, rows_v, sem = buf
        for i in range(CB):
            r0 = i * L_TOT
            zero = jnp.zeros((LANES,), jnp.float32)

            c0, c1 = zero, zero
            for off, lo in ((0, 0), (16, 0), (32, 0), (34, 14)):
                wv = w_v[i, pl.ds(off, LANES)]
                for jl in range(lo, LANES):
                    j = off + jl
                    w = wv[jl]
                    c0 = c0 + rows_v[r0 + j, 0:16] * w
                    c1 = c1 + rows_v[r0 + j, 16:32] * w

            s0, s1 = zero, zero
            for kk in range(L_ITM):
                r = r0 + L_CTX + kk
                s0 = s0 + rows_v[r, 0:16]
                s1 = s1 + rows_v[r, 16:32]

            p = c0 * s0 + c1 * s1
            for sh in (8, 4, 2, 1):
                p = p + jnp.take(p, lane_iota ^ sh)
            dots = jnp.where(lane_iota == (k * CB + i), p, dots)
        return dots

    fire(0, bufs[0])

    def quad_body(q, _):
        c0 = 4 * q
        dots = jnp.zeros((LANES,), jnp.float32)

        fire(c0 + 1, bufs[1])
        drain(bufs[0])
        dots = compute(c0, 0, bufs[0], dots)

        fire(c0 + 2, bufs[0])
        drain(bufs[1])
        dots = compute(c0 + 1, 1, bufs[1], dots)

        fire(c0 + 3, bufs[1])
        drain(bufs[0])
        dots = compute(c0 + 2, 2, bufs[0], dots)

        @pl.when(q + 1 < NQ)
        def _():
            fire(c0 + 4, bufs[0])

        drain(bufs[1])
        dots = compute(c0 + 3, 3, bufs[1], dots)

        out_v[pl.ds(q * LANES, LANES)] = dots
        return 0

    lax.fori_loop(0, NQ, quad_body, 0)
    pltpu.sync_copy(out_v, out_hbm.at[pl.ds(base_b, BW)])


def kernel(ctx, itm, pos, ctx_v, embed1_weight):
    del pos
    table = jnp.pad(embed1_weight, ((0, 0), (0, DP - D)))
    run = pl.kernel(
        _sc_body,
        out_type=jax.ShapeDtypeStruct((B,), jnp.float32),
        mesh=plsc.VectorSubcoreMesh(core_axis_name="c", subcore_axis_name="s",
                                    num_cores=NC, num_subcores=NS),
        scratch_types=[
            pltpu.VMEM((CB, L_CTX), jnp.int32),
            pltpu.VMEM((CB, L_ITM), jnp.int32),
            pltpu.VMEM((CB, L_CTX), jnp.float32),
            pltpu.VMEM((CB * L_TOT, DP), jnp.float32),
            pltpu.VMEM((CB, L_CTX), jnp.int32),
            pltpu.VMEM((CB, L_ITM), jnp.int32),
            pltpu.VMEM((CB, L_CTX), jnp.float32),
            pltpu.VMEM((CB * L_TOT, DP), jnp.float32),
            pltpu.VMEM((BW,), jnp.float32),
            pltpu.SemaphoreType.DMA,
            pltpu.SemaphoreType.DMA,
        ],
        compiler_params=pltpu.CompilerParams(use_tc_tiling_on_sc=False),
    )
    return run(ctx, itm, ctx_v, table)


# speedup vs baseline: 1.1063x; 1.1063x over previous
import jax
import jax.numpy as jnp
from jax import lax
from jax.experimental import pallas as pl
from jax.experimental.pallas import tpu as pltpu
from jax.experimental.pallas import tpu_sc as plsc

B = 16384
D = 32
NROW = 1000000
L_CTX = 50
L_ITM = 20
L_TOT = L_CTX + L_ITM
NC = 2
NS = 16
NW = NC * NS
BW = B // NW
CB = 16
NCHUNK = BW // CB
LANES = 16

RB = 128
QB = RB // 4
NFULL = NROW // RB
TAIL = NROW - NFULL * RB
NSTEP = (NFULL + NW - 1) // NW


def _detile_body(tab_hbm, out_hbm, va, vb, vc):
    wid = lax.axis_index("s") * NC + lax.axis_index("c")

    def step(s, _):
        chunk = s * NW + wid

        @pl.when(chunk < NFULL)
        def _():
            r0 = pl.multiple_of(chunk * RB, RB)
            pltpu.sync_copy(tab_hbm.at[pl.ds(r0, RB)], va)
            for q in range(QB):
                for t in range(4):
                    r = q * 4 + t
                    vc[q, pl.ds(t * D, 16)] = va[r, 0:16]
                    vc[q, pl.ds(t * D + 16, 16)] = va[r, 16:32]
            pltpu.sync_copy(vc,
                            out_hbm.at[pl.ds(pl.multiple_of(chunk * QB, QB),
                                             QB)])
        return 0

    lax.fori_loop(0, NSTEP, step, 0)

    @pl.when(wid == 0)
    def _():
        r0 = NFULL * RB
        pltpu.sync_copy(tab_hbm.at[pl.ds(r0, TAIL)], vb)
        for q in range(TAIL // 4):
            for t in range(4):
                r = q * 4 + t
                vc[q, pl.ds(t * D, 16)] = vb[r, 0:16]
                vc[q, pl.ds(t * D + 16, 16)] = vb[r, 16:32]
        pltpu.sync_copy(vc.at[pl.ds(0, TAIL // 4)],
                        out_hbm.at[pl.ds(r0 // 4, TAIL // 4)])


def _sc_body(ctx_hbm, itm_hbm, w_hbm, table_hbm, out_hbm,
             cidx0, iidx0, w0, rows0, cidx1, iidx1, w1, rows1,
             out_v, sem0, sem1):
    wid = lax.axis_index("s") * NC + lax.axis_index("c")
    base_b = wid * BW
    bufs = ((cidx0, iidx0, w0, rows0, sem0),
            (cidx1, iidx1, w1, rows1, sem1))

    def fire(c, buf):
        cidx_v, iidx_v, w_v, rows_v, sem = buf
        b0 = base_b + c * CB
        pltpu.sync_copy(ctx_hbm.at[pl.ds(b0, CB)], cidx_v)
        pltpu.sync_copy(itm_hbm.at[pl.ds(b0, CB)], iidx_v)
        pltpu.sync_copy(w_hbm.at[pl.ds(b0, CB)], w_v)
        for i in range(CB):
            pltpu.async_copy(table_hbm.at[cidx_v.at[i]],
                             rows_v.at[pl.ds(i * L_TOT, L_CTX)], sem)
            pltpu.async_copy(table_hbm.at[iidx_v.at[i]],
                             rows_v.at[pl.ds(i * L_TOT + L_CTX, L_ITM)], sem)

    def drain(buf):
        cidx_v, iidx_v, w_v, rows_v, sem = buf
        for i in range(CB):
            pltpu.make_async_copy(table_hbm.at[cidx_v.at[i]],
                                  rows_v.at[pl.ds(i * L_TOT, L_CTX)],
                                  sem).wait()
            pltpu.make_async_copy(table_hbm.at[iidx_v.at[i]],
                                  rows_v.at[pl.ds(i * L_TOT + L_CTX, L_ITM)],
                                  sem).wait()

    def compute(c, buf):
        cidx_v, iidx_v, w_v, rows_v, sem = buf
        lane_iota = lax.iota(jnp.int32, LANES)

        def batch_body(i, dots):
            r0 = i * L_TOT
            zero = jnp.zeros((LANES,), jnp.float32)

            c0, c1 = zero, zero
            for off, lo in ((0, 0), (16, 0), (32, 0), (34, 14)):
                wv = w_v[i, pl.ds(off, LANES)]
                for jl in range(lo, LANES):
                    j = off + jl
                    w = wv[jl]
                    c0 = c0 + rows_v[r0 + j, 0:16] * w
                    c1 = c1 + rows_v[r0 + j, 16:32] * w

            s0, s1 = zero, zero
            for k in range(L_ITM):
                r = r0 + L_CTX + k
                s0 = s0 + rows_v[r, 0:16]
                s1 = s1 + rows_v[r, 16:32]

            p = c0 * s0 + c1 * s1
            for sh in (8, 4, 2, 1):
                p = p + jnp.take(p, lane_iota ^ sh)
            return jnp.where(lane_iota == i, p, dots)

        dots = lax.fori_loop(0, CB, batch_body,
                             jnp.zeros((LANES,), jnp.float32))
        out_v[pl.ds(c * CB, CB)] = dots

    fire(0, bufs[0])

    def pair_body(h, _):
        c0 = 2 * h
        fire(c0 + 1, bufs[1])
        drain(bufs[0])
        compute(c0, bufs[0])

        @pl.when(h + 1 < NCHUNK // 2)
        def _():
            fire(c0 + 2, bufs[0])

        drain(bufs[1])
        compute(c0 + 1, bufs[1])
        return 0

    lax.fori_loop(0, NCHUNK // 2, pair_body, 0)
    pltpu.sync_copy(out_v, out_hbm.at[pl.ds(base_b, BW)])


def kernel(ctx, itm, pos, ctx_v, embed1_weight):
    del pos
    detile = pl.kernel(
        _detile_body,
        out_type=jax.ShapeDtypeStruct((NROW // 4, 128), jnp.float32),
        mesh=plsc.VectorSubcoreMesh(core_axis_name="c", subcore_axis_name="s",
                                    num_cores=NC, num_subcores=NS),
        scratch_types=[
            pltpu.VMEM((RB, D), jnp.float32),
            pltpu.VMEM((TAIL, D), jnp.float32),
            pltpu.VMEM((QB, 128), jnp.float32),
        ],
        compiler_params=pltpu.CompilerParams(use_tc_tiling_on_sc=True),
    )
    table4 = detile(embed1_weight)
    table = jnp.reshape(table4, (NROW, D))

    run = pl.kernel(
        _sc_body,
        out_type=jax.ShapeDtypeStruct((B,), jnp.float32),
        mesh=plsc.VectorSubcoreMesh(core_axis_name="c", subcore_axis_name="s",
                                    num_cores=NC, num_subcores=NS),
        scratch_types=[
            pltpu.VMEM((CB, L_CTX), jnp.int32),
            pltpu.VMEM((CB, L_ITM), jnp.int32),
            pltpu.VMEM((CB, L_CTX), jnp.float32),
            pltpu.VMEM((CB * L_TOT, D), jnp.float32),
            pltpu.VMEM((CB, L_CTX), jnp.int32),
            pltpu.VMEM((CB, L_ITM), jnp.int32),
            pltpu.VMEM((CB, L_CTX), jnp.float32),
            pltpu.VMEM((CB * L_TOT, D), jnp.float32),
            pltpu.VMEM((BW,), jnp.float32),
            pltpu.SemaphoreType.DMA,
            pltpu.SemaphoreType.DMA,
        ],
        compiler_params=pltpu.CompilerParams(use_tc_tiling_on_sc=False),
    )
    return run(ctx, itm, ctx_v, table)


# speedup vs baseline: 1.5588x; 1.4090x over previous
import jax
import jax.numpy as jnp
from jax import lax
from jax.experimental import pallas as pl
from jax.experimental.pallas import tpu as pltpu
from jax.experimental.pallas import tpu_sc as plsc

B = 16384
D = 32
L_CTX = 50
L_ITM = 20
L_TOT = L_CTX + L_ITM
NC = 2
NS = 16
NW = NC * NS
BW = B // NW
CB = 16
NCHUNK = BW // CB
LANES = 16
WPAD = 4 * LANES


def _sc_body(idx_hbm, w_hbm, table_hbm, out_hbm,
             idx0, w0, rows0, idx1, w1, rows1, out_v, sem0, sem1):
    wid = lax.axis_index("s") * NC + lax.axis_index("c")
    base_b = wid * BW
    bufs = ((idx0, w0, rows0, sem0), (idx1, w1, rows1, sem1))

    def fire(c, buf):
        idx_v, w_v, rows_v, sem = buf
        b0 = base_b + c * CB
        pltpu.sync_copy(idx_hbm.at[pl.ds(b0, CB)], idx_v)
        pltpu.sync_copy(w_hbm.at[pl.ds(b0, CB)], w_v)
        for i in range(CB):
            pltpu.async_copy(table_hbm.at[idx_v.at[i]],
                             rows_v.at[pl.ds(i * L_TOT, L_TOT)], sem)

    def drain(buf):
        idx_v, w_v, rows_v, sem = buf
        for i in range(CB):
            pltpu.make_async_copy(table_hbm.at[idx_v.at[i]],
                                  rows_v.at[pl.ds(i * L_TOT, L_TOT)],
                                  sem).wait()

    def compute(c, buf):
        idx_v, w_v, rows_v, sem = buf
        lane_iota = lax.iota(jnp.int32, LANES)

        def batch_body(i, dots):
            r0 = i * L_TOT
            zero = jnp.zeros((LANES,), jnp.float32)

            c0, c1 = zero, zero
            for g in range(4):
                wv = w_v[i, pl.ds(g * LANES, LANES)]
                for jl in range(LANES if g < 3 else L_CTX - 3 * LANES):
                    j = g * LANES + jl
                    w = wv[jl]
                    c0 = c0 + rows_v[r0 + j, 0:16] * w
                    c1 = c1 + rows_v[r0 + j, 16:32] * w

            s0, s1 = zero, zero
            for k in range(L_ITM):
                r = r0 + L_CTX + k
                s0 = s0 + rows_v[r, 0:16]
                s1 = s1 + rows_v[r, 16:32]

            p = c0 * s0 + c1 * s1
            for sh in (8, 4, 2, 1):
                p = p + jnp.take(p, lane_iota ^ sh)
            return jnp.where(lane_iota == i, p, dots)

        dots = lax.fori_loop(0, CB, batch_body,
                             jnp.zeros((LANES,), jnp.float32))
        out_v[pl.ds(c * CB, CB)] = dots

    fire(0, bufs[0])

    def pair_body(h, _):
        c0 = 2 * h
        fire(c0 + 1, bufs[1])
        drain(bufs[0])
        compute(c0, bufs[0])

        @pl.when(h + 1 < NCHUNK // 2)
        def _():
            fire(c0 + 2, bufs[0])

        drain(bufs[1])
        compute(c0 + 1, bufs[1])
        return 0

    lax.fori_loop(0, NCHUNK // 2, pair_body, 0)
    pltpu.sync_copy(out_v, out_hbm.at[pl.ds(base_b, BW)])


def kernel(ctx, itm, pos, ctx_v, embed1_weight):
    del pos
    all_idx = jnp.concatenate([ctx, itm], axis=1)
    w_pad = jnp.pad(ctx_v, ((0, 0), (0, WPAD - L_CTX)))

    run = pl.kernel(
        _sc_body,
        out_type=jax.ShapeDtypeStruct((B,), jnp.float32),
        mesh=plsc.VectorSubcoreMesh(core_axis_name="c", subcore_axis_name="s",
                                    num_cores=NC, num_subcores=NS),
        scratch_types=[
            pltpu.VMEM((CB, L_TOT), jnp.int32),
            pltpu.VMEM((CB, WPAD), jnp.float32),
            pltpu.VMEM((CB * L_TOT, D), jnp.float32),
            pltpu.VMEM((CB, L_TOT), jnp.int32),
            pltpu.VMEM((CB, WPAD), jnp.float32),
            pltpu.VMEM((CB * L_TOT, D), jnp.float32),
            pltpu.VMEM((BW,), jnp.float32),
            pltpu.SemaphoreType.DMA,
            pltpu.SemaphoreType.DMA,
        ],
        compiler_params=pltpu.CompilerParams(use_tc_tiling_on_sc=False),
    )
    return run(all_idx, w_pad, embed1_weight)
